# baseline (speedup 1.0000x reference)
import numpy as np
import jax
import jax.numpy as jnp
from jax import lax
from jax.experimental import pallas as pl
from jax.experimental.pallas import tpu as pltpu
from jax.experimental.pallas import tpu_sc as plsc

_S = 8192
_I = 1000
_C = 50
_G = 20
_D = 128
_BS = 1024
_NW = 32
_BW = _S // _NW

_SEG = np.arange(_I) // _G
_MSUM_NP = (_SEG[:, None] == np.arange(_C)[None, :]).astype(np.float32)
_MEXP_NP = _MSUM_NP.T.copy()
_MSEL_NP = (np.arange(_C)[:, None] * _G ==
            np.arange(_BS)[None, :]).astype(np.float32)


def _sc_gather_body(t1_hbm, t2_hbm, idx_hbm, o1_hbm, o2_hbm,
                    idx_v, r1_v, r2_v, s1, s2):
    wid = lax.axis_index("s") * 2 + lax.axis_index("c")
    base = wid * _BW
    pltpu.sync_copy(idx_hbm.at[pl.ds(base, _BW)], idx_v)
    g1 = pltpu.async_copy(t1_hbm.at[idx_v], r1_v, s1)
    g2 = pltpu.async_copy(t2_hbm.at[idx_v], r2_v, s2)
    g1.wait()
    w1 = pltpu.async_copy(r1_v, o1_hbm.at[pl.ds(base, _BW)], s1)
    g2.wait()
    w2 = pltpu.async_copy(r2_v, o2_hbm.at[pl.ds(base, _BW)], s2)
    w1.wait()
    w2.wait()


def _sc_gather(t1, t2, idx):
    mesh = plsc.VectorSubcoreMesh(core_axis_name="c", subcore_axis_name="s")
    k = pl.kernel(
        _sc_gather_body,
        out_type=[
            jax.ShapeDtypeStruct((_S, _D), jnp.float32),
            jax.ShapeDtypeStruct((_S, _D), jnp.float32),
        ],
        mesh=mesh,
        scratch_types=[
            pltpu.VMEM((_BW,), jnp.int32),
            pltpu.VMEM((_BW, _D), jnp.float32),
            pltpu.VMEM((_BW, _D), jnp.float32),
            pltpu.SemaphoreType.DMA,
            pltpu.SemaphoreType.DMA,
        ],
    )
    return k(t1, t2, idx)


def _hi_lo(x):
    hi = x.astype(jnp.bfloat16)
    lo = (x - hi.astype(jnp.float32)).astype(jnp.bfloat16)
    return hi, lo


def _exact_sel_dot(m, x):
    hi, lo = _hi_lo(x)
    return (jnp.dot(m, hi, preferred_element_type=jnp.float32)
            + jnp.dot(m, lo, preferred_element_type=jnp.float32))


def _tc_body(tu_ref, tc_ref, aT_ref, acT_ref, lam_ref,
             msum_ref, mexp_ref, msel_ref, out_ref,
             aTs_ref, lsec_ref):
    f32 = jnp.float32
    bf16 = jnp.bfloat16

    @pl.when(pl.program_id(0) == 0)
    def _prologue():
        il_hi, il_lo = _hi_lo(1.0 / lam_ref[...])
        invl = (jnp.dot(il_hi, mexp_ref[...], preferred_element_type=f32)
                + jnp.dot(il_lo, mexp_ref[...], preferred_element_type=f32))
        aTs = (aT_ref[...] * invl).astype(bf16)
        aTs_ref[...] = aTs
        msel = msel_ref[...].astype(bf16)
        tus = _exact_sel_dot(msel, tu_ref[...])
        tcs = _exact_sel_dot(msel, tc_ref[...])
        ys = jnp.dot(tus.astype(bf16), aTs, preferred_element_type=f32)
        ssum = jnp.dot(jnp.exp(ys).astype(bf16), msum_ref[...],
                       preferred_element_type=f32)
        inc = jnp.log(ssum)
        w = jnp.dot(tcs.astype(bf16), acT_ref[...],
                    preferred_element_type=f32)
        logit = w + lam_ref[...] * inc
        lsec_ref[...] = jnp.log(jnp.sum(jnp.exp(logit), axis=0, keepdims=True))

    tu = tu_ref[...].astype(bf16)
    ys = jnp.dot(tu, aTs_ref[...], preferred_element_type=f32)
    ssum = jnp.dot(jnp.exp(ys).astype(bf16), msum_ref[...],
                   preferred_element_type=f32)
    inc = jnp.log(ssum)
    w = jnp.dot(tc_ref[...].astype(bf16), acT_ref[...],
                preferred_element_type=f32)
    a = (lam_ref[...] - 1.0) * inc + w - lsec_ref[...]
    mu = jnp.mean(a, axis=1, keepdims=True)
    aexp = jnp.dot((a - mu).astype(bf16), mexp_ref[...],
                   preferred_element_type=f32)
    out_ref[...] = (ys + mu) + aexp


def _tc_grid_args():
    full = lambda b: (0, 0)
    in_specs = [
        pl.BlockSpec((_BS, _D), lambda b: (b, 0)),
        pl.BlockSpec((_BS, _D), lambda b: (b, 0)),
        pl.BlockSpec((_D, _I), full),
        pl.BlockSpec((_D, _C), full),
        pl.BlockSpec((1, _C), full),
        pl.BlockSpec((_I, _C), full),
        pl.BlockSpec((_C, _I), full),
        pl.BlockSpec((_C, _BS), full),
    ]
    return dict(
        grid=(_S // _BS,),
        in_specs=in_specs,
        out_specs=pl.BlockSpec((_BS, _I), lambda b: (b, 0)),
        out_shape=jax.ShapeDtypeStruct((_S, _I), jnp.float32),
        scratch_shapes=[pltpu.VMEM((_D, _I), jnp.bfloat16),
                        pltpu.VMEM((1, _C), jnp.float32)],
    )


def kernel(user_index, theta_user_item, alpha_item, theta_user_cat,
           alpha_category, lambda_weight):
    idx = user_index.astype(jnp.int32)
    tu_g, tc_g = _sc_gather(theta_user_item, theta_user_cat, idx)
    lam = lambda_weight.reshape(1, _C)
    aT = alpha_item.T
    acT = alpha_category.T.astype(jnp.bfloat16)
    msum = jnp.asarray(_MSUM_NP).astype(jnp.bfloat16)
    mexp = jnp.asarray(_MEXP_NP).astype(jnp.bfloat16)
    msel = jnp.asarray(_MSEL_NP)
    return pl.pallas_call(_tc_body, **_tc_grid_args())(
        tu_g, tc_g, aT, acT, lam, msum, mexp, msel)
